# baseline (speedup 1.0000x reference)
import functools

import jax
import jax.numpy as jnp
from jax import lax
from jax.experimental import pallas as pl
from jax.experimental.pallas import tpu as pltpu
from jax.experimental.pallas import tpu_sc as plsc

N = 10000
E = 320000
D = 128
NH = 2
DH = D // NH
NC, NS, L = 2, 16, 16
NW = NC * NS
EPW = E // NW
K = 128
EPA = 10240
PAD = NW * EPA - E
NBLK = EPA // K
NSLOT = 4
ZROWS = 125
RPT = N // NS
CH = 2000
NPASS = N // CH
R = 1000
NRB = N // R

_mesh = plsc.VectorSubcoreMesh(core_axis_name="c", subcore_axis_name="s")
_sc_params = pltpu.CompilerParams(
    needs_layout_passes=False, use_tc_tiling_on_sc=False
)


@functools.partial(
    pl.kernel,
    out_type=jax.ShapeDtypeStruct((NRB, NW, R), jnp.float32),
    mesh=_mesh,
    scratch_types=[
        pltpu.VMEM((EPW,), jnp.int32),
        pltpu.VMEM((L * CH,), jnp.float32),
        pltpu.VMEM((N,), jnp.float32),
    ],
    compiler_params=_sc_params,
)
def _deg_kernel(row_hbm, out_hbm, idx_v, hist_v, deg_v):
    c = lax.axis_index("c")
    s = lax.axis_index("s")
    wid = s * NC + c
    pltpu.sync_copy(row_hbm.at[pl.ds(wid * EPW, EPW)], idx_v)
    zero = jnp.zeros((L,), jnp.float32)
    ones = jnp.ones((L,), jnp.float32)
    lane_off = lax.iota(jnp.int32, L) * CH

    for p in range(NPASS):
        base = p * CH

        def zb(i, _):
            for k in range(8):
                hist_v[pl.ds((i * 8 + k) * L, L)] = zero
            return 0

        lax.fori_loop(0, (L * CH) // (L * 8), zb, 0)

        def sb(i, _):
            for k in range(5):
                v = idx_v[pl.ds((i * 5 + k) * L, L)]
                m = (v >= base) & (v < base + CH)
                safe = jnp.where(m, v - base, 0) + lane_off
                plsc.addupdate_scatter(hist_v, [safe], ones, mask=m)
            return 0

        lax.fori_loop(0, EPW // (L * 5), sb, 0)

        def rb(i, _):
            acc = zero
            for l in range(L):
                acc = acc + hist_v[pl.ds(l * CH + i * L, L)]
            deg_v[pl.ds(base + i * L, L)] = acc
            return 0

        lax.fori_loop(0, CH // L, rb, 0)
    for i in range(NRB):
        pltpu.sync_copy(deg_v.at[pl.ds(i * R, R)], out_hbm.at[i, wid])


@functools.partial(
    pl.kernel,
    out_type=jax.ShapeDtypeStruct((NC, N, D), jnp.float32),
    mesh=_mesh,
    scratch_types=[
        pltpu.VMEM((EPA,), jnp.int32),
        pltpu.VMEM((EPA,), jnp.int32),
        pltpu.VMEM((NBLK, K), jnp.int32),
        pltpu.VMEM((NSLOT, K, DH), jnp.float32),
        pltpu.VMEM((ZROWS, DH), jnp.float32),
        pltpu.VMEM_SHARED((N + 8, DH), jnp.float32),
        pltpu.SemaphoreType.DMA((NSLOT,)),
    ],
    compiler_params=_sc_params,
)
def _agg_kernel(
    yf_hbm, col_hbm, row_hbm, out_hbm, col_v, col2_v, row_v, buf_v, zbuf_v, acc_sh, sems
):
    c = lax.axis_index("c")
    s = lax.axis_index("s")
    wid = s * NC + c
    pltpu.sync_copy(row_hbm.at[wid], row_v)
    pltpu.sync_copy(col_hbm.at[wid], col_v)
    zero = jnp.zeros((L,), jnp.float32)

    def zb(i, _):
        zbuf_v[i // (DH // L), pl.ds((i % (DH // L)) * L, L)] = zero
        return 0

    lax.fori_loop(0, ZROWS * (DH // L), zb, 0)

    for h in range(NH):
        def tb(i, _):
            for k in range(5):
                off = (i * 5 + k) * L
                col2_v[pl.ds(off, L)] = col_v[pl.ds(off, L)] * 2 + h
            return 0

        lax.fori_loop(0, EPA // (L * 5), tb, 0)

        def zc(j, _):
            pltpu.sync_copy(zbuf_v, acc_sh.at[pl.ds(s * RPT + j * ZROWS, ZROWS)])
            return 0

        lax.fori_loop(0, RPT // ZROWS, zc, 0)
        plsc.subcore_barrier()

        for k in range(NSLOT):
            pltpu.async_copy(
                yf_hbm.at[col2_v.at[pl.ds(k * K, K)]], buf_v.at[k], sems.at[k]
            )

        def body(jj, _):
            for k in range(NSLOT):
                j = jj * NSLOT + k
                pltpu.make_async_copy(
                    yf_hbm.at[col2_v.at[pl.ds(j * K, K)]], buf_v.at[k], sems.at[k]
                ).wait()
                pltpu.sync_copy(buf_v.at[k], acc_sh.at[row_v.at[j]], add=True)

                @pl.when(j + NSLOT < NBLK)
                def _():
                    pltpu.async_copy(
                        yf_hbm.at[col2_v.at[pl.ds((j + NSLOT) * K, K)]],
                        buf_v.at[k],
                        sems.at[k],
                    )

            return 0

        lax.fori_loop(0, NBLK // NSLOT, body, 0)
        plsc.subcore_barrier()
        pltpu.sync_copy(
            acc_sh.at[pl.ds(s * RPT, RPT)],
            out_hbm.at[c, pl.ds(s * RPT, RPT), pl.ds(h * DH, DH)],
        )


def _prep_body(degp_ref, x_ref, w_ref, dis_ref, y_ref):
    deg = jnp.sum(degp_ref[0], axis=0) + 1.0
    dis = lax.rsqrt(deg)[:, None]
    dis_ref[...] = dis
    y_ref[...] = dis * jnp.dot(
        x_ref[...], w_ref[...], preferred_element_type=jnp.float32
    )


_prep = pl.pallas_call(
    _prep_body,
    grid=(NRB,),
    in_specs=[
        pl.BlockSpec((1, NW, R), lambda i: (i, 0, 0)),
        pl.BlockSpec((R, D), lambda i: (i, 0)),
        pl.BlockSpec((D, D), lambda i: (0, 0)),
    ],
    out_specs=[
        pl.BlockSpec((R, 1), lambda i: (i, 0)),
        pl.BlockSpec((R, D), lambda i: (i, 0)),
    ],
    out_shape=[
        jax.ShapeDtypeStruct((N, 1), jnp.float32),
        jax.ShapeDtypeStruct((N, D), jnp.float32),
    ],
)


def _mid_body(sp_ref, y_ref, dis_ref, w_ref, o_ref):
    ssum = sp_ref[0] + sp_ref[1] + y_ref[...]
    dis = dis_ref[...]
    h = jnp.tanh(dis * ssum) * 5.0
    o_ref[...] = dis * jnp.dot(h, w_ref[...], preferred_element_type=jnp.float32)


_mid = pl.pallas_call(
    _mid_body,
    grid=(NRB,),
    in_specs=[
        pl.BlockSpec((NC, R, D), lambda i: (0, i, 0)),
        pl.BlockSpec((R, D), lambda i: (i, 0)),
        pl.BlockSpec((R, 1), lambda i: (i, 0)),
        pl.BlockSpec((D, D), lambda i: (0, 0)),
    ],
    out_specs=pl.BlockSpec((R, D), lambda i: (i, 0)),
    out_shape=jax.ShapeDtypeStruct((N, D), jnp.float32),
)


def _fin_body(sp_ref, y_ref, dis_ref, o_ref):
    o_ref[...] = dis_ref[...] * (sp_ref[0] + sp_ref[1] + y_ref[...])


_fin = pl.pallas_call(
    _fin_body,
    grid=(NRB,),
    in_specs=[
        pl.BlockSpec((NC, R, D), lambda i: (0, i, 0)),
        pl.BlockSpec((R, D), lambda i: (i, 0)),
        pl.BlockSpec((R, 1), lambda i: (i, 0)),
    ],
    out_specs=pl.BlockSpec((R, D), lambda i: (i, 0)),
    out_shape=jax.ShapeDtypeStruct((N, D), jnp.float32),
)


def kernel(x, edge_index, W1, W2, W3):
    ei = edge_index.astype(jnp.int32)
    row_flat = ei[0]
    row = jnp.concatenate([row_flat, jnp.full((PAD,), N, jnp.int32)]).reshape(
        NW, NBLK, K
    )
    colw = jnp.concatenate([ei[1], jnp.zeros((PAD,), jnp.int32)]).reshape(NW, EPA)
    degp = _deg_kernel(row_flat)
    dis, y = _prep(degp, x, W1)

    def agg(yy):
        return _agg_kernel(yy.reshape(NH * N, DH), colw, row)

    sp = agg(y)
    y2 = _mid(sp, y, dis, W2)
    sp2 = agg(y2)
    y3 = _mid(sp2, y2, dis, W3)
    sp3 = agg(y3)
    return _fin(sp3, y3, dis)


# speedup vs baseline: 3.8786x; 3.8786x over previous
import functools

import jax
import jax.numpy as jnp
from jax import lax
from jax.experimental import pallas as pl
from jax.experimental.pallas import tpu as pltpu
from jax.experimental.pallas import tpu_sc as plsc

N = 10000
E = 320000
D = 128
NH = 2
DH = D // NH
NC, NS, L = 2, 16, 16
NW = NC * NS
EPW = E // NW
K = 80
NBLK = EPW // K
NSLOT = 5
ZROWS = 125
RPT = N // NS
NCOPY = 4
R = 1000
NRB = N // R

_mesh = plsc.VectorSubcoreMesh(core_axis_name="c", subcore_axis_name="s")
_sc_params = pltpu.CompilerParams(
    needs_layout_passes=False, use_tc_tiling_on_sc=False
)


@functools.partial(
    pl.kernel,
    out_type=jax.ShapeDtypeStruct((NRB, NW, R), jnp.float32),
    mesh=_mesh,
    scratch_types=[
        pltpu.VMEM((EPW,), jnp.int32),
        pltpu.VMEM((NCOPY * N,), jnp.float32),
        pltpu.VMEM((N,), jnp.float32),
    ],
    compiler_params=_sc_params,
)
def _deg_kernel(row_hbm, out_hbm, idx_v, hist_v, deg_v):
    c = lax.axis_index("c")
    s = lax.axis_index("s")
    wid = s * NC + c
    pltpu.sync_copy(row_hbm.at[pl.ds(wid * EPW, EPW)], idx_v)
    zero = jnp.zeros((L,), jnp.float32)
    ones = jnp.ones((L,), jnp.float32)
    lane = lax.iota(jnp.int32, L)
    copy_off = (lane % NCOPY) * N
    gmasks = [lane // NCOPY == g for g in range(L // NCOPY)]

    def zb(i, _):
        for k in range(5):
            hist_v[pl.ds((i * 5 + k) * L, L)] = zero
        return 0

    lax.fori_loop(0, (NCOPY * N) // (L * 5), zb, 0)

    def sb(i, _):
        for k in range(5):
            v = idx_v[pl.ds((i * 5 + k) * L, L)] + copy_off
            for g in range(L // NCOPY):
                plsc.addupdate_scatter(hist_v, [v], ones, mask=gmasks[g])
        return 0

    lax.fori_loop(0, EPW // (L * 5), sb, 0)

    def rb(i, _):
        acc = zero
        for l in range(NCOPY):
            acc = acc + hist_v[pl.ds(l * N + i * L, L)]
        deg_v[pl.ds(i * L, L)] = acc
        return 0

    lax.fori_loop(0, N // L, rb, 0)
    for i in range(NRB):
        pltpu.sync_copy(deg_v.at[pl.ds(i * R, R)], out_hbm.at[i, wid])


@functools.partial(
    pl.kernel,
    out_type=jax.ShapeDtypeStruct((NC, N, D), jnp.float32),
    mesh=_mesh,
    scratch_types=[
        pltpu.VMEM((EPW,), jnp.int32),
        pltpu.VMEM((EPW,), jnp.int32),
        pltpu.VMEM((NBLK, K), jnp.int32),
        pltpu.VMEM((NSLOT, K, DH), jnp.float32),
        pltpu.VMEM((ZROWS, DH), jnp.float32),
        pltpu.VMEM_SHARED((N, DH), jnp.float32),
        pltpu.SemaphoreType.DMA((NSLOT,)),
    ],
    compiler_params=_sc_params,
)
def _agg_kernel(
    yf_hbm, col_hbm, row_hbm, out_hbm, col_v, col2_v, row_v, buf_v, zbuf_v, acc_sh, sems
):
    c = lax.axis_index("c")
    s = lax.axis_index("s")
    wid = s * NC + c
    pltpu.sync_copy(row_hbm.at[wid], row_v)
    pltpu.sync_copy(col_hbm.at[wid], col_v)
    zero = jnp.zeros((L,), jnp.float32)

    def zb(i, _):
        zbuf_v[i // (DH // L), pl.ds((i % (DH // L)) * L, L)] = zero
        return 0

    lax.fori_loop(0, ZROWS * (DH // L), zb, 0)

    for h in range(NH):
        def tb(i, _):
            for k in range(5):
                off = (i * 5 + k) * L
                col2_v[pl.ds(off, L)] = col_v[pl.ds(off, L)] * 2 + h
            return 0

        lax.fori_loop(0, EPW // (L * 5), tb, 0)

        def zc(j, _):
            pltpu.sync_copy(zbuf_v, acc_sh.at[pl.ds(s * RPT + j * ZROWS, ZROWS)])
            return 0

        lax.fori_loop(0, RPT // ZROWS, zc, 0)
        plsc.subcore_barrier()

        for k in range(NSLOT):
            pltpu.async_copy(
                yf_hbm.at[col2_v.at[pl.ds(k * K, K)]], buf_v.at[k], sems.at[k]
            )

        def body(jj, _):
            for k in range(NSLOT):
                j = jj * NSLOT + k
                pltpu.make_async_copy(
                    yf_hbm.at[col2_v.at[pl.ds(j * K, K)]], buf_v.at[k], sems.at[k]
                ).wait()
                pltpu.sync_copy(buf_v.at[k], acc_sh.at[row_v.at[j]], add=True)

                @pl.when(j + NSLOT < NBLK)
                def _():
                    pltpu.async_copy(
                        yf_hbm.at[col2_v.at[pl.ds((j + NSLOT) * K, K)]],
                        buf_v.at[k],
                        sems.at[k],
                    )

            return 0

        lax.fori_loop(0, NBLK // NSLOT, body, 0)
        plsc.subcore_barrier()
        pltpu.sync_copy(
            acc_sh.at[pl.ds(s * RPT, RPT)],
            out_hbm.at[c, pl.ds(s * RPT, RPT), pl.ds(h * DH, DH)],
        )


def _matw_body(x_ref, w_ref, o_ref):
    o_ref[...] = jnp.dot(x_ref[...], w_ref[...], preferred_element_type=jnp.float32)


_matw = pl.pallas_call(
    _matw_body,
    grid=(NRB,),
    in_specs=[
        pl.BlockSpec((R, D), lambda i: (i, 0)),
        pl.BlockSpec((D, D), lambda i: (0, 0)),
    ],
    out_specs=pl.BlockSpec((R, D), lambda i: (i, 0)),
    out_shape=jax.ShapeDtypeStruct((N, D), jnp.float32),
)


def _scale_body(degp_ref, yr_ref, dis_ref, y_ref):
    deg = jnp.sum(degp_ref[0], axis=0) + 1.0
    dis = lax.rsqrt(deg)[:, None]
    dis_ref[...] = dis
    y_ref[...] = dis * yr_ref[...]


_scale = pl.pallas_call(
    _scale_body,
    grid=(NRB,),
    in_specs=[
        pl.BlockSpec((1, NW, R), lambda i: (i, 0, 0)),
        pl.BlockSpec((R, D), lambda i: (i, 0)),
    ],
    out_specs=[
        pl.BlockSpec((R, 1), lambda i: (i, 0)),
        pl.BlockSpec((R, D), lambda i: (i, 0)),
    ],
    out_shape=[
        jax.ShapeDtypeStruct((N, 1), jnp.float32),
        jax.ShapeDtypeStruct((N, D), jnp.float32),
    ],
)


def _mid_body(sp_ref, y_ref, dis_ref, w_ref, o_ref):
    ssum = sp_ref[0] + sp_ref[1] + y_ref[...]
    dis = dis_ref[...]
    h = jnp.tanh(dis * ssum) * 5.0
    o_ref[...] = dis * jnp.dot(h, w_ref[...], preferred_element_type=jnp.float32)


_mid = pl.pallas_call(
    _mid_body,
    grid=(NRB,),
    in_specs=[
        pl.BlockSpec((NC, R, D), lambda i: (0, i, 0)),
        pl.BlockSpec((R, D), lambda i: (i, 0)),
        pl.BlockSpec((R, 1), lambda i: (i, 0)),
        pl.BlockSpec((D, D), lambda i: (0, 0)),
    ],
    out_specs=pl.BlockSpec((R, D), lambda i: (i, 0)),
    out_shape=jax.ShapeDtypeStruct((N, D), jnp.float32),
)


def _fin_body(sp_ref, y_ref, dis_ref, o_ref):
    o_ref[...] = dis_ref[...] * (sp_ref[0] + sp_ref[1] + y_ref[...])


_fin = pl.pallas_call(
    _fin_body,
    grid=(NRB,),
    in_specs=[
        pl.BlockSpec((NC, R, D), lambda i: (0, i, 0)),
        pl.BlockSpec((R, D), lambda i: (i, 0)),
        pl.BlockSpec((R, 1), lambda i: (i, 0)),
    ],
    out_specs=pl.BlockSpec((R, D), lambda i: (i, 0)),
    out_shape=jax.ShapeDtypeStruct((N, D), jnp.float32),
)


def kernel(x, edge_index, W1, W2, W3):
    ei = edge_index.astype(jnp.int32)
    row_flat = ei[0]
    row = row_flat.reshape(NW, NBLK, K)
    colw = ei[1].reshape(NW, EPW)
    degp = _deg_kernel(row_flat)
    y_raw = _matw(x, W1)
    dis, y = _scale(degp, y_raw)

    def agg(yy):
        return _agg_kernel(yy.reshape(NH * N, DH), colw, row)

    sp = agg(y)
    y2 = _mid(sp, y, dis, W2)
    sp2 = agg(y2)
    y3 = _mid(sp2, y2, dis, W3)
    sp3 = agg(y3)
    return _fin(sp3, y3, dis)
